# Initial kernel scaffold; baseline (speedup 1.0000x reference)
import functools

import jax
import jax.numpy as jnp
from jax import lax
from jax.experimental import pallas as pl
from jax.experimental.pallas import tpu as pltpu
from jax.experimental.pallas import tpu_sc as plsc

N_NODES = 10000
N_EDGES = 320000
D = 128
L = 16
NC = 2
NS = 16
NW = NC * NS
CHUNK = 128
NCHUNK = N_EDGES // CHUNK
TRIPS = (NCHUNK + NW - 1) // NW
ROWS_PER_TILE = N_NODES // NS
ZROWS = 125


def _sc_partials(x, edge_index, edge_weight):
    mesh = plsc.VectorSubcoreMesh(
        core_axis_name="c", subcore_axis_name="s", num_cores=NC, num_subcores=NS
    )

    @functools.partial(
        pl.kernel,
        out_type=jax.ShapeDtypeStruct((NC, N_NODES, D), jnp.float32),
        mesh=mesh,
        scratch_types=[
            pltpu.VMEM_SHARED((N_NODES, D), jnp.float32),
            pltpu.VMEM((CHUNK,), jnp.int32),
            pltpu.VMEM((CHUNK,), jnp.int32),
            pltpu.VMEM((CHUNK,), jnp.float32),
            pltpu.VMEM((CHUNK, D), jnp.float32),
            pltpu.VMEM((ZROWS, D), jnp.float32),
            pltpu.SemaphoreType.DMA,
        ],
    )
    def k(x_hbm, edge_hbm, w_hbm, parts_hbm, acc, idx_s, idx_d, wbuf, rows, zbuf, sem):
        cid = lax.axis_index("c")
        sid = lax.axis_index("s")
        wid = sid * NC + cid

        zeros = jnp.zeros((L,), jnp.float32)

        def zfill(r, _):
            for j in range(D // L):
                zbuf[r, pl.ds(j * L, L)] = zeros
            return 0

        lax.fori_loop(0, ZROWS, zfill, 0)
        base_row = sid * ROWS_PER_TILE
        for kk in range(ROWS_PER_TILE // ZROWS):
            pltpu.sync_copy(zbuf, acc.at[pl.ds(base_row + kk * ZROWS, ZROWS)])
        plsc.subcore_barrier()

        def chunk_body(i, _):
            c_idx = i * NW + wid

            @pl.when(c_idx < NCHUNK)
            def _():
                base = c_idx * CHUNK
                pltpu.sync_copy(edge_hbm.at[1, pl.ds(base, CHUNK)], idx_s)
                pltpu.sync_copy(edge_hbm.at[0, pl.ds(base, CHUNK)], idx_d)
                pltpu.sync_copy(w_hbm.at[pl.ds(base, CHUNK)], wbuf)
                pltpu.async_copy(x_hbm.at[idx_s], rows, sem).wait()

                def scale(e, _):
                    ws = wbuf[e]
                    for j in range(D // L):
                        sl = pl.ds(j * L, L)
                        rows[e, sl] = rows[e, sl] * ws
                    return 0

                lax.fori_loop(0, CHUNK, scale, 0)
                pltpu.sync_copy(rows, acc.at[idx_d], add=True)

            return 0

        lax.fori_loop(0, TRIPS, chunk_body, 0)
        plsc.subcore_barrier()

        pltpu.sync_copy(
            acc.at[pl.ds(base_row, ROWS_PER_TILE)],
            parts_hbm.at[cid, pl.ds(base_row, ROWS_PER_TILE)],
        )

    return k(x, edge_index, edge_weight)


def _combine(p0, p1):
    def body(a_ref, b_ref, o_ref):
        o_ref[...] = jnp.clip(a_ref[...] + b_ref[...], -20.0, 20.0)

    blk = 500
    spec = pl.BlockSpec((blk, D), lambda i: (i, 0))
    return pl.pallas_call(
        body,
        grid=(N_NODES // blk,),
        in_specs=[spec, spec],
        out_specs=spec,
        out_shape=jax.ShapeDtypeStruct((N_NODES, D), jnp.float32),
    )(p0, p1)


def kernel(t, x, edge_index, edge_weight):
    parts = _sc_partials(x, edge_index, edge_weight)
    return _combine(parts[0], parts[1])


# speedup vs baseline: 5.3716x; 5.3716x over previous
import functools

import jax
import jax.numpy as jnp
from jax import lax
from jax.experimental import pallas as pl
from jax.experimental.pallas import tpu as pltpu
from jax.experimental.pallas import tpu_sc as plsc

N_NODES = 10000
N_EDGES = 320000
D = 128
L = 16
NC = 2
NS = 16
NW = NC * NS
CHUNK = 128
NCHUNK = N_EDGES // CHUNK
TRIPS = (NCHUNK + NW - 1) // NW
ROWS_LO = 624
ROWS_HI = N_NODES - 15 * ROWS_LO
ZROWS = 16


def _sc_partials(x, src, dst, edge_weight):
    mesh = plsc.VectorSubcoreMesh(
        core_axis_name="c", subcore_axis_name="s", num_cores=NC, num_subcores=NS
    )

    @functools.partial(
        pl.kernel,
        out_type=jax.ShapeDtypeStruct((NC, N_NODES, D), jnp.float32),
        mesh=mesh,
        scratch_types=[
            pltpu.VMEM_SHARED((N_NODES, D), jnp.float32),
            pltpu.VMEM((CHUNK,), jnp.int32),
            pltpu.VMEM((CHUNK,), jnp.int32),
            pltpu.VMEM((CHUNK,), jnp.float32),
            pltpu.VMEM((CHUNK, D), jnp.float32),
            pltpu.VMEM((ZROWS, D), jnp.float32),
            pltpu.SemaphoreType.DMA,
        ],
    )
    def k(x_hbm, src_hbm, dst_hbm, w_hbm, parts_hbm, acc, idx_s, idx_d, wbuf,
          rows, zbuf, sem):
        cid = lax.axis_index("c")
        sid = lax.axis_index("s")
        wid = sid * NC + cid
        base_row = sid * ROWS_LO

        zeros = jnp.zeros((L,), jnp.float32)
        for r in range(ZROWS):
            for j in range(D // L):
                zbuf[r, pl.ds(j * L, L)] = zeros

        def zcopy(kk, _):
            pltpu.sync_copy(zbuf, acc.at[pl.ds(base_row + kk * ZROWS, ZROWS)])
            return 0

        n_owned = jnp.where(sid == NS - 1, ROWS_HI, ROWS_LO)
        lax.fori_loop(0, n_owned // ZROWS, zcopy, 0)
        plsc.subcore_barrier()

        def chunk_body(i, _):
            c_idx = i * NW + wid

            @pl.when(c_idx < NCHUNK)
            def _():
                base = c_idx * CHUNK
                pltpu.sync_copy(src_hbm.at[pl.ds(base, CHUNK)], idx_s)
                pltpu.sync_copy(dst_hbm.at[pl.ds(base, CHUNK)], idx_d)
                pltpu.sync_copy(w_hbm.at[pl.ds(base, CHUNK)], wbuf)
                pltpu.async_copy(x_hbm.at[idx_s], rows, sem).wait()

                def scale(g, _):
                    wg = wbuf[pl.ds(g * L, L)]
                    for ee in range(L):
                        e = g * L + ee
                        ws = wg[ee]
                        for j in range(D // L):
                            sl = pl.ds(j * L, L)
                            rows[e, sl] = rows[e, sl] * ws
                    return 0

                lax.fori_loop(0, CHUNK // L, scale, 0)
                pltpu.sync_copy(rows, acc.at[idx_d], add=True)

            return 0

        lax.fori_loop(0, TRIPS, chunk_body, 0)
        plsc.subcore_barrier()

        @pl.when(sid < NS - 1)
        def _():
            pltpu.sync_copy(
                acc.at[pl.ds(base_row, ROWS_LO)],
                parts_hbm.at[cid, pl.ds(base_row, ROWS_LO)],
            )

        @pl.when(sid == NS - 1)
        def _():
            pltpu.sync_copy(
                acc.at[pl.ds(15 * ROWS_LO, ROWS_HI)],
                parts_hbm.at[cid, pl.ds(15 * ROWS_LO, ROWS_HI)],
            )

    return k(x, src, dst, edge_weight)


def _combine(p0, p1):
    def body(a_ref, b_ref, o_ref):
        o_ref[...] = jnp.clip(a_ref[...] + b_ref[...], -20.0, 20.0)

    blk = 1000
    spec = pl.BlockSpec((blk, D), lambda i: (i, 0))
    return pl.pallas_call(
        body,
        grid=(N_NODES // blk,),
        in_specs=[spec, spec],
        out_specs=spec,
        out_shape=jax.ShapeDtypeStruct((N_NODES, D), jnp.float32),
    )(p0, p1)


def kernel(t, x, edge_index, edge_weight):
    parts = _sc_partials(x, edge_index[1], edge_index[0], edge_weight)
    return _combine(parts[0], parts[1])
